# baseline (speedup 1.0000x reference)
import functools

import jax
import jax.numpy as jnp
from jax import lax
from jax.experimental import pallas as pl
from jax.experimental.pallas import tpu as pltpu
from jax.experimental.pallas import tpu_sc as plsc

N = 10000
E = 160000
IN_DIM = 1024
HID = 256
BN_SCALE = 1.0 / (1.0 + 1e-5) ** 0.5

_NSC = 2
_NTILE = 16
_K = 128
_NCH = E // _K
_NPAD = 10112
_ROWS_PER_TILE = _NPAD // _NTILE
_CPT = 80
_CH = _CPT // 2
_EPAD = _NTILE * _CPT * _K
_SINK = _NPAD - 1

_PREC = jax.lax.Precision.HIGHEST


def _dotT(a, w):
    return lax.dot_general(a, w, (((1,), (1,)), ((), ())),
                           preferred_element_type=jnp.float32,
                           precision=_PREC)


def _emb_body(x_ref, w_ref, b_ref, o_ref):
    o_ref[...] = _dotT(x_ref[...], w_ref[...]) + b_ref[...]


def _emb(x, w, b2):
    R = 1000
    return pl.pallas_call(
        _emb_body,
        grid=(N // R,),
        in_specs=[
            pl.BlockSpec((R, IN_DIM), lambda i: (i, 0)),
            pl.BlockSpec((HID, IN_DIM), lambda i: (0, 0)),
            pl.BlockSpec((1, HID), lambda i: (0, 0)),
        ],
        out_specs=pl.BlockSpec((R, HID), lambda i: (i, 0)),
        out_shape=jax.ShapeDtypeStruct((N, HID), jnp.float32),
    )(x, w, b2)


def _node_apply_body(h_ref, c0_ref, c1_ref, p0_ref, p1_ref, w_ref, b_ref,
                     o_ref):
    h = h_ref[...]
    deg = jnp.maximum(p0_ref[:, 0:1] + p1_ref[:, 0:1], 1.0)
    dinv = 1.0 / deg
    w = w_ref[...]
    z = (_dotT(h, w[:, 0:HID])
         + _dotT(c0_ref[...] * dinv, w[:, HID:HID + 128])
         + _dotT(c1_ref[...] * dinv, w[:, HID + 128:HID + 256])
         + b_ref[...])
    nrm = jnp.sqrt(jnp.sum(z * z, axis=1, keepdims=True))
    z = z / jnp.maximum(nrm, 1e-12)
    o_ref[...] = h + jnp.maximum(z, 0.0) * BN_SCALE


def _node_apply(h, c0, c1, p0, p1, w, b2):
    R = 1000
    return pl.pallas_call(
        _node_apply_body,
        grid=(N // R,),
        in_specs=[
            pl.BlockSpec((R, HID), lambda i: (i, 0)),
            pl.BlockSpec((R, 128), lambda i: (i, 0)),
            pl.BlockSpec((R, 128), lambda i: (i, 0)),
            pl.BlockSpec((R, 128), lambda i: (i, 0)),
            pl.BlockSpec((R, 128), lambda i: (i, 0)),
            pl.BlockSpec((HID, 2 * HID), lambda i: (0, 0)),
            pl.BlockSpec((1, HID), lambda i: (0, 0)),
        ],
        out_specs=pl.BlockSpec((R, HID), lambda i: (i, 0)),
        out_shape=jax.ShapeDtypeStruct((N, HID), jnp.float32),
    )(h, c0, c1, p0, p1, w, b2)


def _readout_body(h_ref, w0_ref, b0_ref, w1_ref, b1_ref, w2_ref, b2_ref,
                  o_ref):
    y = jnp.maximum(_dotT(h_ref[...], w0_ref[...]) + b0_ref[...], 0.0)
    y = jnp.maximum(_dotT(y, w1_ref[...]) + b1_ref[...], 0.0)
    o_ref[...] = _dotT(y, w2_ref[...]) + b2_ref[...]


def _readout(h, w0, b0, w1, b1, w2, b2):
    R = 1000
    return pl.pallas_call(
        _readout_body,
        grid=(N // R,),
        in_specs=[
            pl.BlockSpec((R, HID), lambda i: (i, 0)),
            pl.BlockSpec((128, HID), lambda i: (0, 0)),
            pl.BlockSpec((1, 128), lambda i: (0, 0)),
            pl.BlockSpec((64, 128), lambda i: (0, 0)),
            pl.BlockSpec((1, 64), lambda i: (0, 0)),
            pl.BlockSpec((2, 64), lambda i: (0, 0)),
            pl.BlockSpec((1, 2), lambda i: (0, 0)),
        ],
        out_specs=pl.BlockSpec((R, 2), lambda i: (i, 0)),
        out_shape=jax.ShapeDtypeStruct((N, 2), jnp.float32),
    )(h, w0, b0, w1, b1, w2, b2)


def _sc_mesh():
    return plsc.VectorSubcoreMesh(core_axis_name="c", subcore_axis_name="s",
                                  num_cores=_NSC, num_subcores=_NTILE)


@functools.cache
def _make_segsum():
    return functools.partial(
        pl.kernel,
        out_type=jax.ShapeDtypeStruct((_NSC, _NPAD, 128), jnp.float32),
        mesh=_sc_mesh(),
        scratch_types=[
            pltpu.VMEM_SHARED((_NPAD, 128), jnp.float32),
            pltpu.VMEM((_CH, _K), jnp.int32),
            pltpu.VMEM((_CH, _K), jnp.int32),
            pltpu.VMEM((_K, 128), jnp.float32),
            pltpu.VMEM((_K, 128), jnp.float32),
            pltpu.SemaphoreType.DMA,
            pltpu.SemaphoreType.DMA,
        ],
    )(_segsum_body)


def _segsum(h2, srcx2, dst3, zeros):
    return _make_segsum()(h2, srcx2, dst3, zeros)


def _segsum_body(h2_hbm, srcx2_hbm, dst3_hbm, zeros_hbm, out_hbm,
                 acc, gidx, didx, rows0, rows1, sem0, sem1):
    c = lax.axis_index("c")
    s = lax.axis_index("s")
    r0 = s * _ROWS_PER_TILE
    pltpu.sync_copy(zeros_hbm.at[pl.ds(r0, _ROWS_PER_TILE)],
                    acc.at[pl.ds(r0, _ROWS_PER_TILE)])
    plsc.subcore_barrier()

    for hf in range(2):
        pltpu.sync_copy(srcx2_hbm.at[s, c, pl.ds(hf * _CH, _CH)], gidx)
        pltpu.sync_copy(dst3_hbm.at[s, pl.ds(hf * _CH, _CH)], didx)
        pltpu.async_copy(h2_hbm.at[gidx.at[0]], rows0, sem0)

        def body(j, carry):
            i0 = 2 * j
            i1 = i0 + 1
            i2 = i0 + 2
            pltpu.async_copy(h2_hbm.at[gidx.at[i1]], rows1, sem1)
            pltpu.make_async_copy(h2_hbm.at[gidx.at[i0]], rows0, sem0).wait()
            pltpu.sync_copy(rows0, acc.at[didx.at[i0]], add=True)

            @pl.when(i2 < _CH)
            def _():
                pltpu.async_copy(h2_hbm.at[gidx.at[i2]], rows0, sem0)

            pltpu.make_async_copy(h2_hbm.at[gidx.at[i1]], rows1, sem1).wait()
            pltpu.sync_copy(rows1, acc.at[didx.at[i1]], add=True)
            return carry

        lax.fori_loop(0, _CH // 2, body, 0)
    plsc.subcore_barrier()
    pltpu.sync_copy(acc.at[pl.ds(r0, _ROWS_PER_TILE)],
                    out_hbm.at[c, pl.ds(r0, _ROWS_PER_TILE)])


@functools.cache
def _make_deg():
    return functools.partial(
        pl.kernel,
        out_type=jax.ShapeDtypeStruct((_NSC, _NPAD, 128), jnp.float32),
        mesh=_sc_mesh(),
        scratch_types=[
            pltpu.VMEM_SHARED((_NPAD, 128), jnp.float32),
            pltpu.VMEM((_CPT, _K), jnp.int32),
            pltpu.VMEM((_K, 128), jnp.float32),
        ],
    )(_deg_body)


def _deg(dst3, ones, zeros):
    return _make_deg()(dst3, ones, zeros)


def _deg_body(dst3_hbm, ones_hbm, zeros_hbm, out_hbm, acc, didx, ones):
    c = lax.axis_index("c")
    s = lax.axis_index("s")
    r0 = s * _ROWS_PER_TILE
    pltpu.sync_copy(zeros_hbm.at[pl.ds(r0, _ROWS_PER_TILE)],
                    acc.at[pl.ds(r0, _ROWS_PER_TILE)])
    pltpu.sync_copy(ones_hbm, ones)
    pltpu.sync_copy(dst3_hbm.at[s], didx)
    plsc.subcore_barrier()
    half = _CPT // _NSC

    def body(j, carry):
        pltpu.sync_copy(ones, acc.at[didx.at[j + half * c]], add=True)
        return carry

    lax.fori_loop(0, half, body, 0)
    plsc.subcore_barrier()
    pltpu.sync_copy(acc.at[pl.ds(r0, _ROWS_PER_TILE)],
                    out_hbm.at[c, pl.ds(r0, _ROWS_PER_TILE)])


_SEG_SCRATCH = [
    pltpu.VMEM_SHARED((_NPAD, 128), jnp.float32),
    pltpu.VMEM((_CH, _K), jnp.int32),
    pltpu.VMEM((_CH, _K), jnp.int32),
    pltpu.VMEM((_K, 128), jnp.float32),
    pltpu.VMEM((_K, 128), jnp.float32),
    pltpu.SemaphoreType.DMA,
    pltpu.SemaphoreType.DMA,
]

_SEG_SCRATCH_W = [
    pltpu.VMEM_SHARED((_NPAD, 128), jnp.float32),
    pltpu.VMEM((_CH, 2 * _K), jnp.int32),
    pltpu.VMEM((2 * _K, 128), jnp.float32),
    pltpu.SemaphoreType.DMA,
]


def _bench_kernel(body, scratch):
    return functools.partial(
        pl.kernel,
        out_type=jax.ShapeDtypeStruct((_NSC, _NPAD, 128), jnp.float32),
        mesh=_sc_mesh(),
        scratch_types=scratch,
    )(body)


def _prolog(zeros_hbm, acc, s):
    r0 = s * _ROWS_PER_TILE
    pltpu.sync_copy(zeros_hbm.at[pl.ds(r0, _ROWS_PER_TILE)],
                    acc.at[pl.ds(r0, _ROWS_PER_TILE)])
    plsc.subcore_barrier()
    return r0


def _epilog(out_hbm, acc, c, r0):
    plsc.subcore_barrier()
    pltpu.sync_copy(acc.at[pl.ds(r0, _ROWS_PER_TILE)],
                    out_hbm.at[c, pl.ds(r0, _ROWS_PER_TILE)])


def _serial_body(h2_hbm, srcx2_hbm, dst3_hbm, zeros_hbm, out_hbm,
                 acc, gidx, didx, rows0, rows1, sem0, sem1):
    c, s = lax.axis_index("c"), lax.axis_index("s")
    r0 = _prolog(zeros_hbm, acc, s)
    for hf in range(2):
        pltpu.sync_copy(srcx2_hbm.at[s, c, pl.ds(hf * _CH, _CH)], gidx)
        pltpu.sync_copy(dst3_hbm.at[s, pl.ds(hf * _CH, _CH)], didx)

        def body(j, carry):
            pltpu.async_copy(h2_hbm.at[gidx.at[j]], rows0, sem0).wait()
            pltpu.sync_copy(rows0, acc.at[didx.at[j]], add=True)
            return carry

        lax.fori_loop(0, _CH, body, 0)
    _epilog(out_hbm, acc, c, r0)


def _gather_body(h2_hbm, srcx2_hbm, dst3_hbm, zeros_hbm, out_hbm,
                 acc, gidx, didx, rows0, rows1, sem0, sem1):
    c, s = lax.axis_index("c"), lax.axis_index("s")
    r0 = _prolog(zeros_hbm, acc, s)
    for hf in range(2):
        pltpu.sync_copy(srcx2_hbm.at[s, c, pl.ds(hf * _CH, _CH)], gidx)

        def body(j, carry):
            pltpu.async_copy(h2_hbm.at[gidx.at[j]], rows0, sem0).wait()
            return carry

        lax.fori_loop(0, _CH, body, 0)
    _epilog(out_hbm, acc, c, r0)


def _scatter_body(h2_hbm, srcx2_hbm, dst3_hbm, zeros_hbm, out_hbm,
                  acc, gidx, didx, rows0, rows1, sem0, sem1):
    c, s = lax.axis_index("c"), lax.axis_index("s")
    r0 = _prolog(zeros_hbm, acc, s)
    for hf in range(2):
        pltpu.sync_copy(dst3_hbm.at[s, pl.ds(hf * _CH, _CH)], didx)

        def body(j, carry):
            pltpu.sync_copy(rows0, acc.at[didx.at[j]], add=True)
            return carry

        lax.fori_loop(0, _CH, body, 0)
    _epilog(out_hbm, acc, c, r0)


_SEG_SCRATCH_A = _SEG_SCRATCH + [
    pltpu.SemaphoreType.DMA,
    pltpu.SemaphoreType.DMA,
]


def _pipe_async_body(h2_hbm, srcx2_hbm, dst3_hbm, zeros_hbm, out_hbm,
                     acc, gidx, didx, rows0, rows1, semg0, semg1,
                     sems0, sems1):
    c, s = lax.axis_index("c"), lax.axis_index("s")
    r0 = _prolog(zeros_hbm, acc, s)
    for hf in range(2):
        pltpu.sync_copy(srcx2_hbm.at[s, c, pl.ds(hf * _CH, _CH)], gidx)
        pltpu.sync_copy(dst3_hbm.at[s, pl.ds(hf * _CH, _CH)], didx)
        pltpu.async_copy(h2_hbm.at[gidx.at[0]], rows0, semg0)
        pltpu.async_copy(h2_hbm.at[gidx.at[1]], rows1, semg1)

        def body(j, carry):
            i0 = 2 * j
            i1 = i0 + 1
            i2 = i0 + 2
            i3 = i0 + 3
            pltpu.make_async_copy(h2_hbm.at[gidx.at[i0]], rows0,
                                  semg0).wait()
            pltpu.async_copy(rows0, acc.at[didx.at[i0]], sems0, add=True)
            pltpu.make_async_copy(h2_hbm.at[gidx.at[i1]], rows1,
                                  semg1).wait()
            pltpu.async_copy(rows1, acc.at[didx.at[i1]], sems1, add=True)
            pltpu.make_async_copy(rows0, acc.at[didx.at[i0]], sems0).wait()

            @pl.when(i2 < _CH)
            def _():
                pltpu.async_copy(h2_hbm.at[gidx.at[i2]], rows0, semg0)

            pltpu.make_async_copy(rows1, acc.at[didx.at[i1]], sems1).wait()

            @pl.when(i3 < _CH)
            def _():
                pltpu.async_copy(h2_hbm.at[gidx.at[i3]], rows1, semg1)

            return carry

        lax.fori_loop(0, _CH // 2, body, 0)
    _epilog(out_hbm, acc, c, r0)


def kernel(x, edge_index, W_emb, b_emb, W0, b0, W1, b1, W2, b2, W3, b3,
           Wm0, bm0, Wm1, bm1, Wm2, bm2):
    src = edge_index[0].astype(jnp.int32)
    dst = edge_index[1].astype(jnp.int32)
    srcp = jnp.concatenate([src, jnp.zeros((_EPAD - E,), jnp.int32)])
    dstp = jnp.concatenate([dst, jnp.full((_EPAD - E,), _SINK, jnp.int32)])
    sch = (2 * srcp).reshape(_NTILE, 1, _CPT, _K)
    srcx2 = jnp.concatenate([sch, sch + 1], axis=1)
    dst3 = dstp.reshape(_NTILE, _CPT, _K)
    zeros128 = jnp.zeros((_NPAD, 128), jnp.float32)
    ones128 = jnp.zeros((_K, 128), jnp.float32).at[:, 0].set(1.0)

    h = _emb(x, W_emb, b_emb.reshape(1, -1))
    h2 = h.reshape(2 * N, 128)
    variants = [
        ("pipelined", _bench_kernel(_segsum_body, _SEG_SCRATCH)),
        ("serial", _bench_kernel(_serial_body, _SEG_SCRATCH)),
        ("gather_only", _bench_kernel(_gather_body, _SEG_SCRATCH)),
        ("scatter_only", _bench_kernel(_scatter_body, _SEG_SCRATCH)),
        ("pipe_async", _bench_kernel(_pipe_async_body, _SEG_SCRATCH_A)),
    ]
    acc_out = jnp.zeros((N, 2), jnp.float32)
    for name, fn in variants:
        for _r in range(2):
            o = fn(h2, srcx2, dst3, zeros128)
            acc_out = acc_out + o[0, :N, :2]
            h2 = h2 + o[0, 0, 0] * 1e-30
    return acc_out + h2[:N, :2]


# speedup vs baseline: 1.0681x; 1.0681x over previous
import functools

import jax
import jax.numpy as jnp
from jax import lax
from jax.experimental import pallas as pl
from jax.experimental.pallas import tpu as pltpu
from jax.experimental.pallas import tpu_sc as plsc

N = 10000
E = 160000
IN_DIM = 1024
HID = 256
BN_SCALE = 1.0 / (1.0 + 1e-5) ** 0.5

_NSC = 2
_NTILE = 16
_K = 128
_NCH = E // _K
_NPAD = 10112
_ROWS_PER_TILE = _NPAD // _NTILE
_CPT = 80
_CH = _CPT // 2
_EPAD = _NTILE * _CPT * _K
_SINK = _NPAD - 1

_PREC = jax.lax.Precision.HIGHEST


def _dotT(a, w):
    return lax.dot_general(a, w, (((1,), (1,)), ((), ())),
                           preferred_element_type=jnp.float32,
                           precision=_PREC)


def _emb_body(x_ref, w_ref, b_ref, o_ref):
    o_ref[...] = _dotT(x_ref[...], w_ref[...]) + b_ref[...]


def _emb(x, w, b2):
    R = 1000
    return pl.pallas_call(
        _emb_body,
        grid=(N // R,),
        in_specs=[
            pl.BlockSpec((R, IN_DIM), lambda i: (i, 0)),
            pl.BlockSpec((HID, IN_DIM), lambda i: (0, 0)),
            pl.BlockSpec((1, HID), lambda i: (0, 0)),
        ],
        out_specs=pl.BlockSpec((R, HID), lambda i: (i, 0)),
        out_shape=jax.ShapeDtypeStruct((N, HID), jnp.float32),
    )(x, w, b2)


def _node_apply_body(h_ref, c0_ref, c1_ref, p0_ref, p1_ref, w_ref, b_ref,
                     o_ref):
    h = h_ref[...]
    deg = jnp.maximum(p0_ref[:, 0:1] + p1_ref[:, 0:1], 1.0)
    dinv = 1.0 / deg
    w = w_ref[...]
    z = (_dotT(h, w[:, 0:HID])
         + _dotT(c0_ref[...] * dinv, w[:, HID:HID + 128])
         + _dotT(c1_ref[...] * dinv, w[:, HID + 128:HID + 256])
         + b_ref[...])
    nrm = jnp.sqrt(jnp.sum(z * z, axis=1, keepdims=True))
    z = z / jnp.maximum(nrm, 1e-12)
    o_ref[...] = h + jnp.maximum(z, 0.0) * BN_SCALE


def _node_apply(h, c0, c1, p0, p1, w, b2):
    R = 1000
    return pl.pallas_call(
        _node_apply_body,
        grid=(N // R,),
        in_specs=[
            pl.BlockSpec((R, HID), lambda i: (i, 0)),
            pl.BlockSpec((R, 128), lambda i: (i, 0)),
            pl.BlockSpec((R, 128), lambda i: (i, 0)),
            pl.BlockSpec((R, 128), lambda i: (i, 0)),
            pl.BlockSpec((R, 128), lambda i: (i, 0)),
            pl.BlockSpec((HID, 2 * HID), lambda i: (0, 0)),
            pl.BlockSpec((1, HID), lambda i: (0, 0)),
        ],
        out_specs=pl.BlockSpec((R, HID), lambda i: (i, 0)),
        out_shape=jax.ShapeDtypeStruct((N, HID), jnp.float32),
    )(h, c0, c1, p0, p1, w, b2)


def _readout_body(h_ref, w0_ref, b0_ref, w1_ref, b1_ref, w2_ref, b2_ref,
                  o_ref):
    y = jnp.maximum(_dotT(h_ref[...], w0_ref[...]) + b0_ref[...], 0.0)
    y = jnp.maximum(_dotT(y, w1_ref[...]) + b1_ref[...], 0.0)
    o_ref[...] = _dotT(y, w2_ref[...]) + b2_ref[...]


def _readout(h, w0, b0, w1, b1, w2, b2):
    R = 1000
    return pl.pallas_call(
        _readout_body,
        grid=(N // R,),
        in_specs=[
            pl.BlockSpec((R, HID), lambda i: (i, 0)),
            pl.BlockSpec((128, HID), lambda i: (0, 0)),
            pl.BlockSpec((1, 128), lambda i: (0, 0)),
            pl.BlockSpec((64, 128), lambda i: (0, 0)),
            pl.BlockSpec((1, 64), lambda i: (0, 0)),
            pl.BlockSpec((2, 64), lambda i: (0, 0)),
            pl.BlockSpec((1, 2), lambda i: (0, 0)),
        ],
        out_specs=pl.BlockSpec((R, 2), lambda i: (i, 0)),
        out_shape=jax.ShapeDtypeStruct((N, 2), jnp.float32),
    )(h, w0, b0, w1, b1, w2, b2)


def _sc_mesh():
    return plsc.VectorSubcoreMesh(core_axis_name="c", subcore_axis_name="s",
                                  num_cores=_NSC, num_subcores=_NTILE)


@functools.cache
def _make_segsum():
    return functools.partial(
        pl.kernel,
        out_type=jax.ShapeDtypeStruct((_NSC, _NPAD, 128), jnp.float32),
        mesh=_sc_mesh(),
        scratch_types=[
            pltpu.VMEM_SHARED((_NPAD, 128), jnp.float32),
            pltpu.VMEM((_CH, _K), jnp.int32),
            pltpu.VMEM((_CH, _K), jnp.int32),
            pltpu.VMEM((_K, 128), jnp.float32),
            pltpu.VMEM((_K, 128), jnp.float32),
            pltpu.SemaphoreType.DMA,
            pltpu.SemaphoreType.DMA,
        ],
    )(_segsum_body)


def _segsum(h2, srcx2, dst3, zeros):
    return _make_segsum()(h2, srcx2, dst3, zeros)


def _segsum_body(h2_hbm, srcx2_hbm, dst3_hbm, zeros_hbm, out_hbm,
                 acc, gidx, didx, rows0, rows1, sem0, sem1):
    c = lax.axis_index("c")
    s = lax.axis_index("s")
    r0 = s * _ROWS_PER_TILE
    pltpu.sync_copy(zeros_hbm.at[pl.ds(r0, _ROWS_PER_TILE)],
                    acc.at[pl.ds(r0, _ROWS_PER_TILE)])
    plsc.subcore_barrier()

    for hf in range(2):
        pltpu.sync_copy(srcx2_hbm.at[s, c, pl.ds(hf * _CH, _CH)], gidx)
        pltpu.sync_copy(dst3_hbm.at[s, pl.ds(hf * _CH, _CH)], didx)
        pltpu.async_copy(h2_hbm.at[gidx.at[0]], rows0, sem0)

        def body(j, carry):
            i0 = 2 * j
            i1 = i0 + 1
            i2 = i0 + 2
            pltpu.async_copy(h2_hbm.at[gidx.at[i1]], rows1, sem1)
            pltpu.make_async_copy(h2_hbm.at[gidx.at[i0]], rows0, sem0).wait()
            pltpu.sync_copy(rows0, acc.at[didx.at[i0]], add=True)

            @pl.when(i2 < _CH)
            def _():
                pltpu.async_copy(h2_hbm.at[gidx.at[i2]], rows0, sem0)

            pltpu.make_async_copy(h2_hbm.at[gidx.at[i1]], rows1, sem1).wait()
            pltpu.sync_copy(rows1, acc.at[didx.at[i1]], add=True)
            return carry

        lax.fori_loop(0, _CH // 2, body, 0)
    plsc.subcore_barrier()
    pltpu.sync_copy(acc.at[pl.ds(r0, _ROWS_PER_TILE)],
                    out_hbm.at[c, pl.ds(r0, _ROWS_PER_TILE)])


@functools.cache
def _make_deg():
    return functools.partial(
        pl.kernel,
        out_type=jax.ShapeDtypeStruct((_NSC, _NPAD, 128), jnp.float32),
        mesh=_sc_mesh(),
        scratch_types=[
            pltpu.VMEM_SHARED((_NPAD, 128), jnp.float32),
            pltpu.VMEM((_CPT, _K), jnp.int32),
            pltpu.VMEM((_K, 128), jnp.float32),
        ],
    )(_deg_body)


def _deg(dst3, ones, zeros):
    return _make_deg()(dst3, ones, zeros)


def _deg_body(dst3_hbm, ones_hbm, zeros_hbm, out_hbm, acc, didx, ones):
    c = lax.axis_index("c")
    s = lax.axis_index("s")
    r0 = s * _ROWS_PER_TILE
    pltpu.sync_copy(zeros_hbm.at[pl.ds(r0, _ROWS_PER_TILE)],
                    acc.at[pl.ds(r0, _ROWS_PER_TILE)])
    pltpu.sync_copy(ones_hbm, ones)
    pltpu.sync_copy(dst3_hbm.at[s], didx)
    plsc.subcore_barrier()
    half = _CPT // _NSC

    def body(j, carry):
        pltpu.sync_copy(ones, acc.at[didx.at[j + half * c]], add=True)
        return carry

    lax.fori_loop(0, half, body, 0)
    plsc.subcore_barrier()
    pltpu.sync_copy(acc.at[pl.ds(r0, _ROWS_PER_TILE)],
                    out_hbm.at[c, pl.ds(r0, _ROWS_PER_TILE)])


_SEG_SCRATCH = [
    pltpu.VMEM_SHARED((_NPAD, 128), jnp.float32),
    pltpu.VMEM((_CH, _K), jnp.int32),
    pltpu.VMEM((_CH, _K), jnp.int32),
    pltpu.VMEM((_K, 128), jnp.float32),
    pltpu.VMEM((_K, 128), jnp.float32),
    pltpu.SemaphoreType.DMA,
    pltpu.SemaphoreType.DMA,
]

_SEG_SCRATCH_W = [
    pltpu.VMEM_SHARED((_NPAD, 128), jnp.float32),
    pltpu.VMEM((_CH, 2 * _K), jnp.int32),
    pltpu.VMEM((2 * _K, 128), jnp.float32),
    pltpu.SemaphoreType.DMA,
]


def _bench_kernel(body, scratch):
    return functools.partial(
        pl.kernel,
        out_type=jax.ShapeDtypeStruct((_NSC, _NPAD, 128), jnp.float32),
        mesh=_sc_mesh(),
        scratch_types=scratch,
    )(body)


def _prolog(zeros_hbm, acc, s):
    r0 = s * _ROWS_PER_TILE
    pltpu.sync_copy(zeros_hbm.at[pl.ds(r0, _ROWS_PER_TILE)],
                    acc.at[pl.ds(r0, _ROWS_PER_TILE)])
    plsc.subcore_barrier()
    return r0


def _epilog(out_hbm, acc, c, r0):
    plsc.subcore_barrier()
    pltpu.sync_copy(acc.at[pl.ds(r0, _ROWS_PER_TILE)],
                    out_hbm.at[c, pl.ds(r0, _ROWS_PER_TILE)])


def _r1_body(h2_hbm, srcx2_hbm, dst3_hbm, zeros_hbm, out_hbm,
             acc, gidx1, didx1, rows0, sem0):
    c, s = lax.axis_index("c"), lax.axis_index("s")
    r0 = _prolog(zeros_hbm, acc, s)

    def body(i, carry):
        pltpu.sync_copy(srcx2_hbm.at[s, c, i], gidx1)
        pltpu.sync_copy(dst3_hbm.at[s, i], didx1)
        pltpu.async_copy(h2_hbm.at[gidx1], rows0, sem0).wait()
        pltpu.sync_copy(rows0, acc.at[didx1], add=True)
        return carry

    lax.fori_loop(0, _CPT, body, 0)
    _epilog(out_hbm, acc, c, r0)


_R1_SCRATCH = [
    pltpu.VMEM_SHARED((_NPAD, 128), jnp.float32),
    pltpu.VMEM((_K,), jnp.int32),
    pltpu.VMEM((_K,), jnp.int32),
    pltpu.VMEM((_K, 128), jnp.float32),
    pltpu.SemaphoreType.DMA,
]


_G4_SCRATCH = [
    pltpu.VMEM((_CPT, _K), jnp.int32),
    pltpu.VMEM((_K, 128), jnp.float32),
    pltpu.VMEM((_K, 128), jnp.float32),
    pltpu.VMEM((_K, 128), jnp.float32),
    pltpu.VMEM((_K, 128), jnp.float32),
    pltpu.SemaphoreType.DMA,
    pltpu.SemaphoreType.DMA,
    pltpu.SemaphoreType.DMA,
    pltpu.SemaphoreType.DMA,
]


def _gather4_body(h2_hbm, srcx2_hbm, dst3_hbm, zeros_hbm, out_hbm,
                  gidx, r0b, r1b, r2b, r3b, s0, s1, s2, s3):
    c, s = lax.axis_index("c"), lax.axis_index("s")
    pltpu.sync_copy(srcx2_hbm.at[s, c], gidx)
    bufs = [(r0b, s0), (r1b, s1), (r2b, s2), (r3b, s3)]
    for b, (rb, sb) in enumerate(bufs):
        pltpu.async_copy(h2_hbm.at[gidx.at[b]], rb, sb)

    def body(j, carry):
        i = 4 * j
        for b, (rb, sb) in enumerate(bufs):
            pltpu.make_async_copy(h2_hbm.at[gidx.at[i + b]], rb, sb).wait()

            @pl.when(i + b + 4 < _CPT)
            def _():
                pltpu.async_copy(h2_hbm.at[gidx.at[i + b + 4]], rb, sb)

        return carry

    lax.fori_loop(0, _CPT // 4, body, 0)
    pltpu.sync_copy(r0b, out_hbm.at[c, pl.ds(s * _ROWS_PER_TILE, _K)])


_G1K_SCRATCH = [
    pltpu.VMEM((2 * _CPT, 64), jnp.int32),
    pltpu.VMEM((64, 256), jnp.float32),
    pltpu.SemaphoreType.DMA,
]


def _gather1k_body(h_hbm, srck_hbm, dst3_hbm, zeros_hbm, out_hbm,
                   gidx, rows, sem):
    c, s = lax.axis_index("c"), lax.axis_index("s")
    pltpu.sync_copy(srck_hbm.at[s, c], gidx)

    def body(j, carry):
        pltpu.async_copy(h_hbm.at[gidx.at[j]], rows, sem).wait()
        return carry

    lax.fori_loop(0, 2 * _CPT, body, 0)
    pltpu.sync_copy(rows.at[pl.ds(0, 64), pl.ds(0, 128)],
                    out_hbm.at[c, pl.ds(s * _ROWS_PER_TILE, 64)])


def _gather_body(h2_hbm, srcx2_hbm, dst3_hbm, zeros_hbm, out_hbm,
                 acc, gidx, didx, rows0, rows1, sem0, sem1):
    c, s = lax.axis_index("c"), lax.axis_index("s")
    r0 = _prolog(zeros_hbm, acc, s)
    for hf in range(2):
        pltpu.sync_copy(srcx2_hbm.at[s, c, pl.ds(hf * _CH, _CH)], gidx)

        def body(j, carry):
            pltpu.async_copy(h2_hbm.at[gidx.at[j]], rows0, sem0).wait()
            return carry

        lax.fori_loop(0, _CH, body, 0)
    _epilog(out_hbm, acc, c, r0)


def _scatter_body(h2_hbm, srcx2_hbm, dst3_hbm, zeros_hbm, out_hbm,
                  acc, gidx, didx, rows0, rows1, sem0, sem1):
    c, s = lax.axis_index("c"), lax.axis_index("s")
    r0 = _prolog(zeros_hbm, acc, s)
    for hf in range(2):
        pltpu.sync_copy(dst3_hbm.at[s, pl.ds(hf * _CH, _CH)], didx)

        def body(j, carry):
            pltpu.sync_copy(rows0, acc.at[didx.at[j]], add=True)
            return carry

        lax.fori_loop(0, _CH, body, 0)
    _epilog(out_hbm, acc, c, r0)


_SEG_SCRATCH_A = _SEG_SCRATCH + [
    pltpu.SemaphoreType.DMA,
    pltpu.SemaphoreType.DMA,
]


def _pipe_async_body(h2_hbm, srcx2_hbm, dst3_hbm, zeros_hbm, out_hbm,
                     acc, gidx, didx, rows0, rows1, semg0, semg1,
                     sems0, sems1):
    c, s = lax.axis_index("c"), lax.axis_index("s")
    r0 = _prolog(zeros_hbm, acc, s)
    for hf in range(2):
        pltpu.sync_copy(srcx2_hbm.at[s, c, pl.ds(hf * _CH, _CH)], gidx)
        pltpu.sync_copy(dst3_hbm.at[s, pl.ds(hf * _CH, _CH)], didx)
        pltpu.async_copy(h2_hbm.at[gidx.at[0]], rows0, semg0)
        pltpu.async_copy(h2_hbm.at[gidx.at[1]], rows1, semg1)

        def body(j, carry):
            i0 = 2 * j
            i1 = i0 + 1
            i2 = i0 + 2
            i3 = i0 + 3
            pltpu.make_async_copy(h2_hbm.at[gidx.at[i0]], rows0,
                                  semg0).wait()
            pltpu.async_copy(rows0, acc.at[didx.at[i0]], sems0, add=True)
            pltpu.make_async_copy(h2_hbm.at[gidx.at[i1]], rows1,
                                  semg1).wait()
            pltpu.async_copy(rows1, acc.at[didx.at[i1]], sems1, add=True)
            pltpu.make_async_copy(rows0, acc.at[didx.at[i0]], sems0).wait()

            @pl.when(i2 < _CH)
            def _():
                pltpu.async_copy(h2_hbm.at[gidx.at[i2]], rows0, semg0)

            pltpu.make_async_copy(rows1, acc.at[didx.at[i1]], sems1).wait()

            @pl.when(i3 < _CH)
            def _():
                pltpu.async_copy(h2_hbm.at[gidx.at[i3]], rows1, semg1)

            return carry

        lax.fori_loop(0, _CH // 2, body, 0)
    _epilog(out_hbm, acc, c, r0)


def kernel(x, edge_index, W_emb, b_emb, W0, b0, W1, b1, W2, b2, W3, b3,
           Wm0, bm0, Wm1, bm1, Wm2, bm2):
    src = edge_index[0].astype(jnp.int32)
    dst = edge_index[1].astype(jnp.int32)
    srcp = jnp.concatenate([src, jnp.zeros((_EPAD - E,), jnp.int32)])
    dstp = jnp.concatenate([dst, jnp.full((_EPAD - E,), _SINK, jnp.int32)])
    sch = (2 * srcp).reshape(_NTILE, 1, _CPT, _K)
    srcx2 = jnp.concatenate([sch, sch + 1], axis=1)
    dst3 = dstp.reshape(_NTILE, _CPT, _K)
    zeros128 = jnp.zeros((_NPAD, 128), jnp.float32)
    ones128 = jnp.zeros((_K, 128), jnp.float32).at[:, 0].set(1.0)

    h = _emb(x, W_emb, b_emb.reshape(1, -1))
    h2 = h.reshape(2 * N, 128)
    variants = [
        ("r1", _bench_kernel(_r1_body, _R1_SCRATCH)),
        ("pipelined", _bench_kernel(_segsum_body, _SEG_SCRATCH)),
        ("gather4", _bench_kernel(_gather4_body, _G4_SCRATCH)),
        ("gather1k", _bench_kernel(_gather1k_body, _G1K_SCRATCH)),
    ]
    t64 = srcp.reshape(_NTILE, 2 * _CPT, 64)
    srck = jnp.stack([t64, t64], axis=1)
    acc_out = jnp.zeros((N, 2), jnp.float32)
    for name, fn in variants:
        for _r in range(2):
            if name == "gather1k":
                o = fn(h, srck, dst3, zeros128)
            else:
                o = fn(h2, srcx2, dst3, zeros128)
            acc_out = acc_out + o[0, :N, :2]
            h2 = h2 + o[0, 0, 0] * 1e-30
    return acc_out + h2[:N, :2]


# speedup vs baseline: 1.6498x; 1.5446x over previous
import functools

import jax
import jax.numpy as jnp
from jax import lax
from jax.experimental import pallas as pl
from jax.experimental.pallas import tpu as pltpu
from jax.experimental.pallas import tpu_sc as plsc

N = 10000
E = 160000
IN_DIM = 1024
HID = 256
BN_SCALE = 1.0 / (1.0 + 1e-5) ** 0.5

_NSC = 2
_NTILE = 16
_K = 128
_NCH = E // _K
_NPAD = 10112
_ROWS_PER_TILE = _NPAD // _NTILE
_CPT = 80
_CH = _CPT // 2
_EPAD = _NTILE * _CPT * _K
_SINK = _NPAD - 1

_PREC = jax.lax.Precision.HIGHEST


def _dotT(a, w):
    return lax.dot_general(a, w, (((1,), (1,)), ((), ())),
                           preferred_element_type=jnp.float32,
                           precision=_PREC)


def _emb_body(x_ref, w_ref, b_ref, o_ref):
    o_ref[...] = _dotT(x_ref[...], w_ref[...]) + b_ref[...]


def _emb(x, w, b2):
    R = 1000
    return pl.pallas_call(
        _emb_body,
        grid=(N // R,),
        in_specs=[
            pl.BlockSpec((R, IN_DIM), lambda i: (i, 0)),
            pl.BlockSpec((HID, IN_DIM), lambda i: (0, 0)),
            pl.BlockSpec((1, HID), lambda i: (0, 0)),
        ],
        out_specs=pl.BlockSpec((R, HID), lambda i: (i, 0)),
        out_shape=jax.ShapeDtypeStruct((N, HID), jnp.float32),
    )(x, w, b2)


def _node_apply_body(h_ref, c0_ref, c1_ref, p0_ref, p1_ref, w_ref, b_ref,
                     o_ref):
    h = h_ref[...]
    deg = jnp.maximum(p0_ref[:, 0:1] + p1_ref[:, 0:1], 1.0)
    dinv = 1.0 / deg
    w = w_ref[...]
    z = (_dotT(h, w[:, 0:HID])
         + _dotT(c0_ref[...] * dinv, w[:, HID:HID + 128])
         + _dotT(c1_ref[...] * dinv, w[:, HID + 128:HID + 256])
         + b_ref[...])
    nrm = jnp.sqrt(jnp.sum(z * z, axis=1, keepdims=True))
    z = z / jnp.maximum(nrm, 1e-12)
    o_ref[...] = h + jnp.maximum(z, 0.0) * BN_SCALE


def _node_apply(h, c0, c1, p0, p1, w, b2):
    R = 1000
    return pl.pallas_call(
        _node_apply_body,
        grid=(N // R,),
        in_specs=[
            pl.BlockSpec((R, HID), lambda i: (i, 0)),
            pl.BlockSpec((R, 128), lambda i: (i, 0)),
            pl.BlockSpec((R, 128), lambda i: (i, 0)),
            pl.BlockSpec((R, 128), lambda i: (i, 0)),
            pl.BlockSpec((R, 128), lambda i: (i, 0)),
            pl.BlockSpec((HID, 2 * HID), lambda i: (0, 0)),
            pl.BlockSpec((1, HID), lambda i: (0, 0)),
        ],
        out_specs=pl.BlockSpec((R, HID), lambda i: (i, 0)),
        out_shape=jax.ShapeDtypeStruct((N, HID), jnp.float32),
    )(h, c0, c1, p0, p1, w, b2)


def _readout_body(h_ref, w0_ref, b0_ref, w1_ref, b1_ref, w2_ref, b2_ref,
                  o_ref):
    y = jnp.maximum(_dotT(h_ref[...], w0_ref[...]) + b0_ref[...], 0.0)
    y = jnp.maximum(_dotT(y, w1_ref[...]) + b1_ref[...], 0.0)
    o_ref[...] = _dotT(y, w2_ref[...]) + b2_ref[...]


def _readout(h, w0, b0, w1, b1, w2, b2):
    R = 1000
    return pl.pallas_call(
        _readout_body,
        grid=(N // R,),
        in_specs=[
            pl.BlockSpec((R, HID), lambda i: (i, 0)),
            pl.BlockSpec((128, HID), lambda i: (0, 0)),
            pl.BlockSpec((1, 128), lambda i: (0, 0)),
            pl.BlockSpec((64, 128), lambda i: (0, 0)),
            pl.BlockSpec((1, 64), lambda i: (0, 0)),
            pl.BlockSpec((2, 64), lambda i: (0, 0)),
            pl.BlockSpec((1, 2), lambda i: (0, 0)),
        ],
        out_specs=pl.BlockSpec((R, 2), lambda i: (i, 0)),
        out_shape=jax.ShapeDtypeStruct((N, 2), jnp.float32),
    )(h, w0, b0, w1, b1, w2, b2)


def _sc_mesh():
    return plsc.VectorSubcoreMesh(core_axis_name="c", subcore_axis_name="s",
                                  num_cores=_NSC, num_subcores=_NTILE)


@functools.cache
def _make_segsum():
    return functools.partial(
        pl.kernel,
        out_type=jax.ShapeDtypeStruct((_NSC, _NPAD, 128), jnp.float32),
        mesh=_sc_mesh(),
        scratch_types=[
            pltpu.VMEM_SHARED((_NPAD, 128), jnp.float32),
            pltpu.VMEM((_CH, _K), jnp.int32),
            pltpu.VMEM((_CH, _K), jnp.int32),
            pltpu.VMEM((_K, 128), jnp.float32),
            pltpu.VMEM((_K, 128), jnp.float32),
            pltpu.SemaphoreType.DMA,
            pltpu.SemaphoreType.DMA,
        ],
    )(_segsum_body)


def _segsum(h2, srcx2, dst3, zeros):
    return _make_segsum()(h2, srcx2, dst3, zeros)


def _segsum_body(h2_hbm, srcx2_hbm, dst3_hbm, zeros_hbm, out_hbm,
                 acc, gidx, didx, rows0, rows1, sem0, sem1):
    c = lax.axis_index("c")
    s = lax.axis_index("s")
    r0 = s * _ROWS_PER_TILE
    pltpu.sync_copy(zeros_hbm.at[pl.ds(r0, _ROWS_PER_TILE)],
                    acc.at[pl.ds(r0, _ROWS_PER_TILE)])
    plsc.subcore_barrier()

    for hf in range(2):
        pltpu.sync_copy(srcx2_hbm.at[s, c, pl.ds(hf * _CH, _CH)], gidx)
        pltpu.sync_copy(dst3_hbm.at[s, pl.ds(hf * _CH, _CH)], didx)
        pltpu.async_copy(h2_hbm.at[gidx.at[0]], rows0, sem0)

        def body(j, carry):
            i0 = 2 * j
            i1 = i0 + 1
            i2 = i0 + 2
            pltpu.async_copy(h2_hbm.at[gidx.at[i1]], rows1, sem1)
            pltpu.make_async_copy(h2_hbm.at[gidx.at[i0]], rows0, sem0).wait()
            pltpu.sync_copy(rows0, acc.at[didx.at[i0]], add=True)

            @pl.when(i2 < _CH)
            def _():
                pltpu.async_copy(h2_hbm.at[gidx.at[i2]], rows0, sem0)

            pltpu.make_async_copy(h2_hbm.at[gidx.at[i1]], rows1, sem1).wait()
            pltpu.sync_copy(rows1, acc.at[didx.at[i1]], add=True)
            return carry

        lax.fori_loop(0, _CH // 2, body, 0)
    plsc.subcore_barrier()
    pltpu.sync_copy(acc.at[pl.ds(r0, _ROWS_PER_TILE)],
                    out_hbm.at[c, pl.ds(r0, _ROWS_PER_TILE)])


@functools.cache
def _make_deg():
    return functools.partial(
        pl.kernel,
        out_type=jax.ShapeDtypeStruct((_NSC, _NPAD, 128), jnp.float32),
        mesh=_sc_mesh(),
        scratch_types=[
            pltpu.VMEM_SHARED((_NPAD, 128), jnp.float32),
            pltpu.VMEM((_CPT, _K), jnp.int32),
            pltpu.VMEM((_K, 128), jnp.float32),
        ],
    )(_deg_body)


def _deg(dst3, ones, zeros):
    return _make_deg()(dst3, ones, zeros)


def _deg_body(dst3_hbm, ones_hbm, zeros_hbm, out_hbm, acc, didx, ones):
    c = lax.axis_index("c")
    s = lax.axis_index("s")
    r0 = s * _ROWS_PER_TILE
    pltpu.sync_copy(zeros_hbm.at[pl.ds(r0, _ROWS_PER_TILE)],
                    acc.at[pl.ds(r0, _ROWS_PER_TILE)])
    pltpu.sync_copy(ones_hbm, ones)
    pltpu.sync_copy(dst3_hbm.at[s], didx)
    plsc.subcore_barrier()
    half = _CPT // _NSC

    def body(j, carry):
        pltpu.sync_copy(ones, acc.at[didx.at[j + half * c]], add=True)
        return carry

    lax.fori_loop(0, half, body, 0)
    plsc.subcore_barrier()
    pltpu.sync_copy(acc.at[pl.ds(r0, _ROWS_PER_TILE)],
                    out_hbm.at[c, pl.ds(r0, _ROWS_PER_TILE)])


_SEG_SCRATCH = [
    pltpu.VMEM_SHARED((_NPAD, 128), jnp.float32),
    pltpu.VMEM((_CH, _K), jnp.int32),
    pltpu.VMEM((_CH, _K), jnp.int32),
    pltpu.VMEM((_K, 128), jnp.float32),
    pltpu.VMEM((_K, 128), jnp.float32),
    pltpu.SemaphoreType.DMA,
    pltpu.SemaphoreType.DMA,
]

_SEG_SCRATCH_W = [
    pltpu.VMEM_SHARED((_NPAD, 128), jnp.float32),
    pltpu.VMEM((_CH, 2 * _K), jnp.int32),
    pltpu.VMEM((2 * _K, 128), jnp.float32),
    pltpu.SemaphoreType.DMA,
]


def _bench_kernel(body, scratch):
    return functools.partial(
        pl.kernel,
        out_type=jax.ShapeDtypeStruct((_NSC, _NPAD, 128), jnp.float32),
        mesh=_sc_mesh(),
        scratch_types=scratch,
    )(body)


def _prolog(zeros_hbm, acc, s):
    r0 = s * _ROWS_PER_TILE
    pltpu.sync_copy(zeros_hbm.at[pl.ds(r0, _ROWS_PER_TILE)],
                    acc.at[pl.ds(r0, _ROWS_PER_TILE)])
    plsc.subcore_barrier()
    return r0


def _epilog(out_hbm, acc, c, r0):
    plsc.subcore_barrier()
    pltpu.sync_copy(acc.at[pl.ds(r0, _ROWS_PER_TILE)],
                    out_hbm.at[c, pl.ds(r0, _ROWS_PER_TILE)])


def _r1_body(h2_hbm, srcx_hbm, dst_hbm, zeros_hbm, out_hbm,
             acc, gidx1, didx1, rows0, sem0):
    c, s = lax.axis_index("c"), lax.axis_index("s")
    r0 = _prolog(zeros_hbm, acc, s)

    def body(i, carry):
        ch = s + i * _NTILE

        @pl.when(ch < _NCH)
        def _():
            e0 = ch * _K
            pltpu.sync_copy(srcx_hbm.at[pl.ds((2 * ch + c) * _K, _K)], gidx1)
            pltpu.sync_copy(dst_hbm.at[pl.ds(e0, _K)], didx1)
            pltpu.async_copy(h2_hbm.at[gidx1], rows0, sem0).wait()
            pltpu.sync_copy(rows0, acc.at[didx1], add=True)

        return carry

    lax.fori_loop(0, (_NCH + _NTILE - 1) // _NTILE, body, 0)
    _epilog(out_hbm, acc, c, r0)


_R1_SCRATCH = [
    pltpu.VMEM_SHARED((_NPAD, 128), jnp.float32),
    pltpu.VMEM((_K,), jnp.int32),
    pltpu.VMEM((_K,), jnp.int32),
    pltpu.VMEM((_K, 128), jnp.float32),
    pltpu.SemaphoreType.DMA,
]


_G4_SCRATCH = [
    pltpu.VMEM((_CPT, _K), jnp.int32),
    pltpu.VMEM((_K, 128), jnp.float32),
    pltpu.VMEM((_K, 128), jnp.float32),
    pltpu.VMEM((_K, 128), jnp.float32),
    pltpu.VMEM((_K, 128), jnp.float32),
    pltpu.SemaphoreType.DMA,
    pltpu.SemaphoreType.DMA,
    pltpu.SemaphoreType.DMA,
    pltpu.SemaphoreType.DMA,
]


def _gather4_body(h2_hbm, srcx2_hbm, dst3_hbm, zeros_hbm, out_hbm,
                  gidx, r0b, r1b, r2b, r3b, s0, s1, s2, s3):
    c, s = lax.axis_index("c"), lax.axis_index("s")
    pltpu.sync_copy(srcx2_hbm.at[s, c], gidx)
    bufs = [(r0b, s0), (r1b, s1), (r2b, s2), (r3b, s3)]
    for b, (rb, sb) in enumerate(bufs):
        pltpu.async_copy(h2_hbm.at[gidx.at[b]], rb, sb)

    def body(j, carry):
        i = 4 * j
        for b, (rb, sb) in enumerate(bufs):
            pltpu.make_async_copy(h2_hbm.at[gidx.at[i + b]], rb, sb).wait()

            @pl.when(i + b + 4 < _CPT)
            def _():
                pltpu.async_copy(h2_hbm.at[gidx.at[i + b + 4]], rb, sb)

        return carry

    lax.fori_loop(0, _CPT // 4, body, 0)
    pltpu.sync_copy(r0b, out_hbm.at[c, pl.ds(s * _ROWS_PER_TILE, _K)])


_G1K_SCRATCH = [
    pltpu.VMEM((2 * _CPT, 64), jnp.int32),
    pltpu.VMEM((64, 256), jnp.float32),
    pltpu.SemaphoreType.DMA,
]


def _gather1k_body(h_hbm, srck_hbm, dst3_hbm, zeros_hbm, out_hbm,
                   gidx, rows, sem):
    c, s = lax.axis_index("c"), lax.axis_index("s")
    pltpu.sync_copy(srck_hbm.at[s, c], gidx)

    def body(j, carry):
        pltpu.async_copy(h_hbm.at[gidx.at[j]], rows, sem).wait()
        return carry

    lax.fori_loop(0, 2 * _CPT, body, 0)
    pltpu.sync_copy(rows.at[pl.ds(0, 64), pl.ds(0, 128)],
                    out_hbm.at[c, pl.ds(s * _ROWS_PER_TILE, 64)])


def _gather_body(h2_hbm, srcx2_hbm, dst3_hbm, zeros_hbm, out_hbm,
                 acc, gidx, didx, rows0, rows1, sem0, sem1):
    c, s = lax.axis_index("c"), lax.axis_index("s")
    r0 = _prolog(zeros_hbm, acc, s)
    for hf in range(2):
        pltpu.sync_copy(srcx2_hbm.at[s, c, pl.ds(hf * _CH, _CH)], gidx)

        def body(j, carry):
            pltpu.async_copy(h2_hbm.at[gidx.at[j]], rows0, sem0).wait()
            return carry

        lax.fori_loop(0, _CH, body, 0)
    _epilog(out_hbm, acc, c, r0)


def _scatter_body(h2_hbm, srcx2_hbm, dst3_hbm, zeros_hbm, out_hbm,
                  acc, gidx, didx, rows0, rows1, sem0, sem1):
    c, s = lax.axis_index("c"), lax.axis_index("s")
    r0 = _prolog(zeros_hbm, acc, s)
    for hf in range(2):
        pltpu.sync_copy(dst3_hbm.at[s, pl.ds(hf * _CH, _CH)], didx)

        def body(j, carry):
            pltpu.sync_copy(rows0, acc.at[didx.at[j]], add=True)
            return carry

        lax.fori_loop(0, _CH, body, 0)
    _epilog(out_hbm, acc, c, r0)


_SEG_SCRATCH_A = _SEG_SCRATCH + [
    pltpu.SemaphoreType.DMA,
    pltpu.SemaphoreType.DMA,
]


def _pipe_async_body(h2_hbm, srcx2_hbm, dst3_hbm, zeros_hbm, out_hbm,
                     acc, gidx, didx, rows0, rows1, semg0, semg1,
                     sems0, sems1):
    c, s = lax.axis_index("c"), lax.axis_index("s")
    r0 = _prolog(zeros_hbm, acc, s)
    for hf in range(2):
        pltpu.sync_copy(srcx2_hbm.at[s, c, pl.ds(hf * _CH, _CH)], gidx)
        pltpu.sync_copy(dst3_hbm.at[s, pl.ds(hf * _CH, _CH)], didx)
        pltpu.async_copy(h2_hbm.at[gidx.at[0]], rows0, semg0)
        pltpu.async_copy(h2_hbm.at[gidx.at[1]], rows1, semg1)

        def body(j, carry):
            i0 = 2 * j
            i1 = i0 + 1
            i2 = i0 + 2
            i3 = i0 + 3
            pltpu.make_async_copy(h2_hbm.at[gidx.at[i0]], rows0,
                                  semg0).wait()
            pltpu.async_copy(rows0, acc.at[didx.at[i0]], sems0, add=True)
            pltpu.make_async_copy(h2_hbm.at[gidx.at[i1]], rows1,
                                  semg1).wait()
            pltpu.async_copy(rows1, acc.at[didx.at[i1]], sems1, add=True)
            pltpu.make_async_copy(rows0, acc.at[didx.at[i0]], sems0).wait()

            @pl.when(i2 < _CH)
            def _():
                pltpu.async_copy(h2_hbm.at[gidx.at[i2]], rows0, semg0)

            pltpu.make_async_copy(rows1, acc.at[didx.at[i1]], sems1).wait()

            @pl.when(i3 < _CH)
            def _():
                pltpu.async_copy(h2_hbm.at[gidx.at[i3]], rows1, semg1)

            return carry

        lax.fori_loop(0, _CH // 2, body, 0)
    _epilog(out_hbm, acc, c, r0)


def kernel(x, edge_index, W_emb, b_emb, W0, b0, W1, b1, W2, b2, W3, b3,
           Wm0, bm0, Wm1, bm1, Wm2, bm2):
    src = edge_index[0].astype(jnp.int32)
    dst = edge_index[1].astype(jnp.int32)
    srcp = jnp.concatenate([src, jnp.zeros((_EPAD - E,), jnp.int32)])
    dstp = jnp.concatenate([dst, jnp.full((_EPAD - E,), _SINK, jnp.int32)])
    sch = (2 * srcp).reshape(_NTILE, 1, _CPT, _K)
    srcx2 = jnp.concatenate([sch, sch + 1], axis=1)
    dst3 = dstp.reshape(_NTILE, _CPT, _K)
    zeros128 = jnp.zeros((_NPAD, 128), jnp.float32)
    ones128 = jnp.zeros((_K, 128), jnp.float32).at[:, 0].set(1.0)

    h = _emb(x, W_emb, b_emb.reshape(1, -1))
    h2 = h.reshape(2 * N, 128)
    variants = [
        ("r1", _bench_kernel(_r1_body, _R1_SCRATCH)),
        ("pipelined", _bench_kernel(_segsum_body, _SEG_SCRATCH)),
        ("pipe_strided", _bench_kernel(_segsum_body, _SEG_SCRATCH)),
    ]
    s2f = (2 * src).reshape(_NCH, 1, _K)
    srcxf = jnp.concatenate([s2f, s2f + 1], axis=1).reshape(-1)
    perm = (jnp.arange(_CPT)[None, :] * _NTILE
            + jnp.arange(_NTILE)[:, None]).reshape(-1)
    dst3s = dstp.reshape(_NTILE * _CPT, _K)[perm].reshape(
        _NTILE, _CPT, _K)
    sch_s = (2 * srcp).reshape(_NTILE * _CPT, 1, _K)[perm].reshape(
        _NTILE, 1, _CPT, _K)
    srcx2s = jnp.concatenate([sch_s, sch_s + 1], axis=1)
    acc_out = jnp.zeros((N, 2), jnp.float32)
    for name, fn in variants:
        for _r in range(2):
            if name == "r1":
                o = fn(h2, srcxf, dst, zeros128)
            elif name == "pipe_strided":
                o = fn(h2, srcx2s, dst3s, zeros128)
            else:
                o = fn(h2, srcx2, dst3, zeros128)
            acc_out = acc_out + o[0, :N, :2]
            h2 = h2 + o[0, 0, 0] * 1e-30
    return acc_out + h2[:N, :2]


# speedup vs baseline: 2.1577x; 1.3079x over previous
import functools

import jax
import jax.numpy as jnp
from jax import lax
from jax.experimental import pallas as pl
from jax.experimental.pallas import tpu as pltpu
from jax.experimental.pallas import tpu_sc as plsc

N = 10000
E = 160000
IN_DIM = 1024
HID = 256
BN_SCALE = 1.0 / (1.0 + 1e-5) ** 0.5

_NSC = 2
_NTILE = 16
_K = 128
_NCH = E // _K
_NPAD = 10112
_ROWS_PER_TILE = _NPAD // _NTILE

_PREC = jax.lax.Precision.HIGHEST


def _dotT(a, w):
    return lax.dot_general(a, w, (((1,), (1,)), ((), ())),
                           preferred_element_type=jnp.float32,
                           precision=_PREC)


def _emb_body(x_ref, w_ref, b_ref, o_ref):
    o_ref[...] = _dotT(x_ref[...], w_ref[...]) + b_ref[...]


def _emb(x, w, b2):
    R = 1000
    return pl.pallas_call(
        _emb_body,
        grid=(N // R,),
        in_specs=[
            pl.BlockSpec((R, IN_DIM), lambda i: (i, 0)),
            pl.BlockSpec((HID, IN_DIM), lambda i: (0, 0)),
            pl.BlockSpec((1, HID), lambda i: (0, 0)),
        ],
        out_specs=pl.BlockSpec((R, HID), lambda i: (i, 0)),
        out_shape=jax.ShapeDtypeStruct((N, HID), jnp.float32),
    )(x, w, b2)


def _node_apply_body(h_ref, c0_ref, c1_ref, p0_ref, p1_ref, w_ref, b_ref,
                     o_ref):
    h = h_ref[...]
    deg = jnp.maximum(p0_ref[0, :, 0:1] + p1_ref[0, :, 0:1], 1.0)
    dinv = 1.0 / deg
    w = w_ref[...]
    z = (_dotT(h, w[:, 0:HID])
         + _dotT(c0_ref[0] * dinv, w[:, HID:HID + 128])
         + _dotT(c1_ref[0] * dinv, w[:, HID + 128:HID + 256])
         + b_ref[...])
    nrm = jnp.sqrt(jnp.sum(z * z, axis=1, keepdims=True))
    z = z / jnp.maximum(nrm, 1e-12)
    o_ref[...] = h + jnp.maximum(z, 0.0) * BN_SCALE


def _node_apply(h, cs, degp, w, b2):
    R = 1000
    return pl.pallas_call(
        _node_apply_body,
        grid=(N // R,),
        in_specs=[
            pl.BlockSpec((R, HID), lambda i: (i, 0)),
            pl.BlockSpec((1, R, 128), lambda i: (0, i, 0)),
            pl.BlockSpec((1, R, 128), lambda i: (1, i, 0)),
            pl.BlockSpec((1, R, 128), lambda i: (0, i, 0)),
            pl.BlockSpec((1, R, 128), lambda i: (1, i, 0)),
            pl.BlockSpec((HID, 2 * HID), lambda i: (0, 0)),
            pl.BlockSpec((1, HID), lambda i: (0, 0)),
        ],
        out_specs=pl.BlockSpec((R, HID), lambda i: (i, 0)),
        out_shape=jax.ShapeDtypeStruct((N, HID), jnp.float32),
    )(h, cs, cs, degp, degp, w, b2)


def _readout_body(h_ref, w0_ref, b0_ref, w1_ref, b1_ref, w2_ref, b2_ref,
                  o_ref):
    y = jnp.maximum(_dotT(h_ref[...], w0_ref[...]) + b0_ref[...], 0.0)
    y = jnp.maximum(_dotT(y, w1_ref[...]) + b1_ref[...], 0.0)
    o_ref[...] = _dotT(y, w2_ref[...]) + b2_ref[...]


def _readout(h, w0, b0, w1, b1, w2, b2):
    R = 1000
    return pl.pallas_call(
        _readout_body,
        grid=(N // R,),
        in_specs=[
            pl.BlockSpec((R, HID), lambda i: (i, 0)),
            pl.BlockSpec((128, HID), lambda i: (0, 0)),
            pl.BlockSpec((1, 128), lambda i: (0, 0)),
            pl.BlockSpec((64, 128), lambda i: (0, 0)),
            pl.BlockSpec((1, 64), lambda i: (0, 0)),
            pl.BlockSpec((2, 64), lambda i: (0, 0)),
            pl.BlockSpec((1, 2), lambda i: (0, 0)),
        ],
        out_specs=pl.BlockSpec((R, 2), lambda i: (i, 0)),
        out_shape=jax.ShapeDtypeStruct((N, 2), jnp.float32),
    )(h, w0, b0, w1, b1, w2, b2)


def _sc_mesh():
    return plsc.VectorSubcoreMesh(core_axis_name="c", subcore_axis_name="s",
                                  num_cores=_NSC, num_subcores=_NTILE)


@functools.cache
def _make_segsum():
    return functools.partial(
        pl.kernel,
        out_type=jax.ShapeDtypeStruct((_NSC, _NPAD, 128), jnp.float32),
        mesh=_sc_mesh(),
        scratch_types=[
            pltpu.VMEM_SHARED((_NPAD, 128), jnp.float32),
            pltpu.VMEM((_K,), jnp.int32),
            pltpu.VMEM((_K,), jnp.int32),
            pltpu.VMEM((_K, 128), jnp.float32),
            pltpu.SemaphoreType.DMA,
        ],
    )(_segsum_body)


def _segsum(h2, srcx, dst, zeros):
    return _make_segsum()(h2, srcx, dst, zeros)


def _segsum_body(h2_hbm, srcx_hbm, dst_hbm, zeros_hbm, out_hbm,
                 acc, gidx, sidx, rows, sem):
    c = lax.axis_index("c")
    s = lax.axis_index("s")
    r0 = s * _ROWS_PER_TILE
    pltpu.sync_copy(zeros_hbm.at[pl.ds(r0, _ROWS_PER_TILE)],
                    acc.at[pl.ds(r0, _ROWS_PER_TILE)])
    plsc.subcore_barrier()

    def body(i, carry):
        ch = s + i * _NTILE

        @pl.when(ch < _NCH)
        def _():
            e0 = ch * _K
            pltpu.sync_copy(srcx_hbm.at[pl.ds((2 * ch + c) * _K, _K)], gidx)
            pltpu.sync_copy(dst_hbm.at[pl.ds(e0, _K)], sidx)
            pltpu.async_copy(h2_hbm.at[gidx], rows, sem).wait()
            pltpu.sync_copy(rows, acc.at[sidx], add=True)

        return carry

    lax.fori_loop(0, (_NCH + _NTILE - 1) // _NTILE, body, 0)
    plsc.subcore_barrier()
    pltpu.sync_copy(acc.at[pl.ds(r0, _ROWS_PER_TILE)],
                    out_hbm.at[c, pl.ds(r0, _ROWS_PER_TILE)])


@functools.cache
def _make_deg():
    return functools.partial(
        pl.kernel,
        out_type=jax.ShapeDtypeStruct((_NSC, _NPAD, 128), jnp.float32),
        mesh=_sc_mesh(),
        scratch_types=[
            pltpu.VMEM_SHARED((_NPAD, 128), jnp.float32),
            pltpu.VMEM((_K,), jnp.int32),
            pltpu.VMEM((_K, 128), jnp.float32),
        ],
    )(_deg_body)


def _deg(dst, ones, zeros):
    return _make_deg()(dst, ones, zeros)


def _deg_body(dst_hbm, ones_hbm, zeros_hbm, out_hbm, acc, sidx, ones):
    c = lax.axis_index("c")
    s = lax.axis_index("s")
    r0 = s * _ROWS_PER_TILE
    pltpu.sync_copy(zeros_hbm.at[pl.ds(r0, _ROWS_PER_TILE)],
                    acc.at[pl.ds(r0, _ROWS_PER_TILE)])
    pltpu.sync_copy(ones_hbm, ones)
    plsc.subcore_barrier()
    half = _NCH // _NSC

    def body(i, carry):
        k = s + i * _NTILE

        @pl.when(k < half)
        def _():
            e0 = (c + _NSC * k) * _K
            pltpu.sync_copy(dst_hbm.at[pl.ds(e0, _K)], sidx)
            pltpu.sync_copy(ones, acc.at[sidx], add=True)

        return carry

    lax.fori_loop(0, (half + _NTILE - 1) // _NTILE, body, 0)
    plsc.subcore_barrier()
    pltpu.sync_copy(acc.at[pl.ds(r0, _ROWS_PER_TILE)],
                    out_hbm.at[c, pl.ds(r0, _ROWS_PER_TILE)])


def kernel(x, edge_index, W_emb, b_emb, W0, b0, W1, b1, W2, b2, W3, b3,
           Wm0, bm0, Wm1, bm1, Wm2, bm2):
    src = edge_index[0].astype(jnp.int32)
    dst = edge_index[1].astype(jnp.int32)
    s2 = (2 * src).reshape(_NCH, 1, _K)
    srcx = jnp.concatenate([s2, s2 + 1], axis=1).reshape(-1)
    zeros128 = jnp.zeros((_NPAD, 128), jnp.float32)
    ones128 = jnp.zeros((_K, 128), jnp.float32).at[:, 0].set(1.0)

    h = _emb(x, W_emb, b_emb.reshape(1, -1))
    degp = _deg(dst, ones128, zeros128)
    for W, b in ((W0, b0), (W1, b1), (W2, b2), (W3, b3)):
        cs = _segsum(h.reshape(2 * N, 128), srcx, dst, zeros128)
        h = _node_apply(h, cs, degp, W, b.reshape(1, -1))
    return _readout(h, Wm0, bm0.reshape(1, -1), Wm1, bm1.reshape(1, -1),
                    Wm2, bm2.reshape(1, -1))


# speedup vs baseline: 2.3487x; 1.0885x over previous
import functools

import jax
import jax.numpy as jnp
from jax import lax
from jax.experimental import pallas as pl
from jax.experimental.pallas import tpu as pltpu
from jax.experimental.pallas import tpu_sc as plsc

N = 10000
E = 160000
IN_DIM = 1024
HID = 256
BN_SCALE = 1.0 / (1.0 + 1e-5) ** 0.5

_NSC = 2
_NTILE = 16
_K = 128
_NCH = E // _K
_NPAD = 10112
_ROWS_PER_TILE = _NPAD // _NTILE

_PREC = jax.lax.Precision.DEFAULT


def _dotT(a, w):
    return lax.dot_general(a, w, (((1,), (1,)), ((), ())),
                           preferred_element_type=jnp.float32,
                           precision=_PREC)


def _emb_body(x_ref, w_ref, b_ref, o_ref):
    o_ref[...] = _dotT(x_ref[...], w_ref[...]) + b_ref[...]


def _emb(x, w, b2):
    R = 1000
    return pl.pallas_call(
        _emb_body,
        grid=(N // R,),
        in_specs=[
            pl.BlockSpec((R, IN_DIM), lambda i: (i, 0)),
            pl.BlockSpec((HID, IN_DIM), lambda i: (0, 0)),
            pl.BlockSpec((1, HID), lambda i: (0, 0)),
        ],
        out_specs=pl.BlockSpec((R, HID), lambda i: (i, 0)),
        out_shape=jax.ShapeDtypeStruct((N, HID), jnp.float32),
    )(x, w, b2)


def _node_apply_body(h_ref, c0_ref, c1_ref, p0_ref, p1_ref, w_ref, b_ref,
                     o_ref):
    h = h_ref[...]
    deg = jnp.maximum(p0_ref[0, :, 0:1] + p1_ref[0, :, 0:1], 1.0)
    dinv = 1.0 / deg
    w = w_ref[...]
    z = (_dotT(h, w[:, 0:HID])
         + _dotT(c0_ref[0] * dinv, w[:, HID:HID + 128])
         + _dotT(c1_ref[0] * dinv, w[:, HID + 128:HID + 256])
         + b_ref[...])
    nrm = jnp.sqrt(jnp.sum(z * z, axis=1, keepdims=True))
    z = z / jnp.maximum(nrm, 1e-12)
    o_ref[...] = h + jnp.maximum(z, 0.0) * BN_SCALE


def _node_apply(h, cs, degp, w, b2):
    R = 1000
    return pl.pallas_call(
        _node_apply_body,
        grid=(N // R,),
        in_specs=[
            pl.BlockSpec((R, HID), lambda i: (i, 0)),
            pl.BlockSpec((1, R, 128), lambda i: (0, i, 0)),
            pl.BlockSpec((1, R, 128), lambda i: (1, i, 0)),
            pl.BlockSpec((1, R, 128), lambda i: (0, i, 0)),
            pl.BlockSpec((1, R, 128), lambda i: (1, i, 0)),
            pl.BlockSpec((HID, 2 * HID), lambda i: (0, 0)),
            pl.BlockSpec((1, HID), lambda i: (0, 0)),
        ],
        out_specs=pl.BlockSpec((R, HID), lambda i: (i, 0)),
        out_shape=jax.ShapeDtypeStruct((N, HID), jnp.float32),
    )(h, cs, cs, degp, degp, w, b2)


def _readout_body(h_ref, w0_ref, b0_ref, w1_ref, b1_ref, w2_ref, b2_ref,
                  o_ref):
    y = jnp.maximum(_dotT(h_ref[...], w0_ref[...]) + b0_ref[...], 0.0)
    y = jnp.maximum(_dotT(y, w1_ref[...]) + b1_ref[...], 0.0)
    o_ref[...] = _dotT(y, w2_ref[...]) + b2_ref[...]


def _readout(h, w0, b0, w1, b1, w2, b2):
    R = 1000
    return pl.pallas_call(
        _readout_body,
        grid=(N // R,),
        in_specs=[
            pl.BlockSpec((R, HID), lambda i: (i, 0)),
            pl.BlockSpec((128, HID), lambda i: (0, 0)),
            pl.BlockSpec((1, 128), lambda i: (0, 0)),
            pl.BlockSpec((64, 128), lambda i: (0, 0)),
            pl.BlockSpec((1, 64), lambda i: (0, 0)),
            pl.BlockSpec((2, 64), lambda i: (0, 0)),
            pl.BlockSpec((1, 2), lambda i: (0, 0)),
        ],
        out_specs=pl.BlockSpec((R, 2), lambda i: (i, 0)),
        out_shape=jax.ShapeDtypeStruct((N, 2), jnp.float32),
    )(h, w0, b0, w1, b1, w2, b2)


def _sc_mesh():
    return plsc.VectorSubcoreMesh(core_axis_name="c", subcore_axis_name="s",
                                  num_cores=_NSC, num_subcores=_NTILE)


@functools.cache
def _make_segsum():
    return functools.partial(
        pl.kernel,
        out_type=jax.ShapeDtypeStruct((_NSC, _NPAD, 128), jnp.float32),
        mesh=_sc_mesh(),
        scratch_types=[
            pltpu.VMEM_SHARED((_NPAD, 128), jnp.float32),
            pltpu.VMEM((_K,), jnp.int32),
            pltpu.VMEM((_K,), jnp.int32),
            pltpu.VMEM((_K, 128), jnp.float32),
            pltpu.SemaphoreType.DMA,
        ],
    )(_segsum_body)


def _segsum(h2, srcx, dst, zeros):
    return _make_segsum()(h2, srcx, dst, zeros)


def _segsum_body(h2_hbm, srcx_hbm, dst_hbm, zeros_hbm, out_hbm,
                 acc, gidx, sidx, rows, sem):
    c = lax.axis_index("c")
    s = lax.axis_index("s")
    r0 = s * _ROWS_PER_TILE
    pltpu.sync_copy(zeros_hbm.at[pl.ds(r0, _ROWS_PER_TILE)],
                    acc.at[pl.ds(r0, _ROWS_PER_TILE)])
    plsc.subcore_barrier()

    def body(i, carry):
        ch = s + i * _NTILE

        @pl.when(ch < _NCH)
        def _():
            e0 = ch * _K
            pltpu.sync_copy(srcx_hbm.at[pl.ds((2 * ch + c) * _K, _K)], gidx)
            pltpu.sync_copy(dst_hbm.at[pl.ds(e0, _K)], sidx)
            pltpu.async_copy(h2_hbm.at[gidx], rows, sem).wait()
            pltpu.sync_copy(rows, acc.at[sidx], add=True)

        return carry

    lax.fori_loop(0, (_NCH + _NTILE - 1) // _NTILE, body, 0)
    plsc.subcore_barrier()
    pltpu.sync_copy(acc.at[pl.ds(r0, _ROWS_PER_TILE)],
                    out_hbm.at[c, pl.ds(r0, _ROWS_PER_TILE)])


@functools.cache
def _make_deg():
    return functools.partial(
        pl.kernel,
        out_type=jax.ShapeDtypeStruct((_NSC, _NPAD, 128), jnp.float32),
        mesh=_sc_mesh(),
        scratch_types=[
            pltpu.VMEM_SHARED((_NPAD, 128), jnp.float32),
            pltpu.VMEM((_K,), jnp.int32),
            pltpu.VMEM((_K, 128), jnp.float32),
        ],
    )(_deg_body)


def _deg(dst, ones, zeros):
    return _make_deg()(dst, ones, zeros)


def _deg_body(dst_hbm, ones_hbm, zeros_hbm, out_hbm, acc, sidx, ones):
    c = lax.axis_index("c")
    s = lax.axis_index("s")
    r0 = s * _ROWS_PER_TILE
    pltpu.sync_copy(zeros_hbm.at[pl.ds(r0, _ROWS_PER_TILE)],
                    acc.at[pl.ds(r0, _ROWS_PER_TILE)])
    pltpu.sync_copy(ones_hbm, ones)
    plsc.subcore_barrier()
    half = _NCH // _NSC

    def body(i, carry):
        k = s + i * _NTILE

        @pl.when(k < half)
        def _():
            e0 = (c + _NSC * k) * _K
            pltpu.sync_copy(dst_hbm.at[pl.ds(e0, _K)], sidx)
            pltpu.sync_copy(ones, acc.at[sidx], add=True)

        return carry

    lax.fori_loop(0, (half + _NTILE - 1) // _NTILE, body, 0)
    plsc.subcore_barrier()
    pltpu.sync_copy(acc.at[pl.ds(r0, _ROWS_PER_TILE)],
                    out_hbm.at[c, pl.ds(r0, _ROWS_PER_TILE)])


def kernel(x, edge_index, W_emb, b_emb, W0, b0, W1, b1, W2, b2, W3, b3,
           Wm0, bm0, Wm1, bm1, Wm2, bm2):
    src = edge_index[0].astype(jnp.int32)
    dst = edge_index[1].astype(jnp.int32)
    s2 = (2 * src).reshape(_NCH, 1, _K)
    srcx = jnp.concatenate([s2, s2 + 1], axis=1).reshape(-1)
    zeros128 = jnp.zeros((_NPAD, 128), jnp.float32)
    ones128 = jnp.zeros((_K, 128), jnp.float32).at[:, 0].set(1.0)

    h = _emb(x, W_emb, b_emb.reshape(1, -1))
    degp = _deg(dst, ones128, zeros128)
    for W, b in ((W0, b0), (W1, b1), (W2, b2), (W3, b3)):
        cs = _segsum(h.reshape(2 * N, 128), srcx, dst, zeros128)
        h = _node_apply(h, cs, degp, W, b.reshape(1, -1))
    return _readout(h, Wm0, bm0.reshape(1, -1), Wm1, bm1.reshape(1, -1),
                    Wm2, bm2.reshape(1, -1))


# speedup vs baseline: 2.4429x; 1.0401x over previous
import functools

import jax
import jax.numpy as jnp
from jax import lax
from jax.experimental import pallas as pl
from jax.experimental.pallas import tpu as pltpu
from jax.experimental.pallas import tpu_sc as plsc

N = 10000
E = 160000
IN_DIM = 1024
HID = 256
BN_SCALE = 1.0 / (1.0 + 1e-5) ** 0.5

_NSC = 2
_NTILE = 16
_K = 128
_NCH = E // _K
_NPAD = 10112
_ROWS_PER_TILE = _NPAD // _NTILE

_PREC = jax.lax.Precision.DEFAULT


def _dotT(a, w):
    return lax.dot_general(a, w, (((1,), (1,)), ((), ())),
                           preferred_element_type=jnp.float32,
                           precision=_PREC)


def _emb_body(x_ref, w_ref, b_ref, o0_ref, o1_ref):
    y = _dotT(x_ref[...], w_ref[...]) + b_ref[...]
    o0_ref[...] = y[:, :128]
    o1_ref[...] = y[:, 128:]


def _emb(x, w, b2):
    R = 1000
    return pl.pallas_call(
        _emb_body,
        grid=(N // R,),
        in_specs=[
            pl.BlockSpec((R, IN_DIM), lambda i: (i, 0)),
            pl.BlockSpec((HID, IN_DIM), lambda i: (0, 0)),
            pl.BlockSpec((1, HID), lambda i: (0, 0)),
        ],
        out_specs=[pl.BlockSpec((R, 128), lambda i: (i, 0)),
                   pl.BlockSpec((R, 128), lambda i: (i, 0))],
        out_shape=[jax.ShapeDtypeStruct((N, 128), jnp.float32),
                   jax.ShapeDtypeStruct((N, 128), jnp.float32)],
    )(x, w, b2)


def _node_apply_body(h0_ref, h1_ref, c0_ref, c1_ref, p0_ref, p1_ref,
                     w_ref, b_ref, o0_ref, o1_ref):
    h0 = h0_ref[...]
    h1 = h1_ref[...]
    deg = jnp.maximum(p0_ref[0, :, 0:1] + p1_ref[0, :, 0:1], 1.0)
    dinv = 1.0 / deg
    w = w_ref[...]
    z = (_dotT(h0, w[:, 0:128])
         + _dotT(h1, w[:, 128:HID])
         + _dotT(c0_ref[0] * dinv, w[:, HID:HID + 128])
         + _dotT(c1_ref[0] * dinv, w[:, HID + 128:HID + 256])
         + b_ref[...])
    nrm = jnp.sqrt(jnp.sum(z * z, axis=1, keepdims=True))
    z = jnp.maximum(z / jnp.maximum(nrm, 1e-12), 0.0) * BN_SCALE
    o0_ref[...] = h0 + z[:, :128]
    o1_ref[...] = h1 + z[:, 128:]


def _node_apply(h0, h1, cs, degp, w, b2):
    R = 1000
    return pl.pallas_call(
        _node_apply_body,
        grid=(N // R,),
        in_specs=[
            pl.BlockSpec((R, 128), lambda i: (i, 0)),
            pl.BlockSpec((R, 128), lambda i: (i, 0)),
            pl.BlockSpec((1, R, 128), lambda i: (0, i, 0)),
            pl.BlockSpec((1, R, 128), lambda i: (1, i, 0)),
            pl.BlockSpec((1, R, 128), lambda i: (0, i, 0)),
            pl.BlockSpec((1, R, 128), lambda i: (1, i, 0)),
            pl.BlockSpec((HID, 2 * HID), lambda i: (0, 0)),
            pl.BlockSpec((1, HID), lambda i: (0, 0)),
        ],
        out_specs=[pl.BlockSpec((R, 128), lambda i: (i, 0)),
                   pl.BlockSpec((R, 128), lambda i: (i, 0))],
        out_shape=[jax.ShapeDtypeStruct((N, 128), jnp.float32),
                   jax.ShapeDtypeStruct((N, 128), jnp.float32)],
    )(h0, h1, cs, cs, degp, degp, w, b2)


def _readout_body(h0_ref, h1_ref, w0_ref, b0_ref, w1_ref, b1_ref, w2_ref,
                  b2_ref, o_ref):
    w0 = w0_ref[...]
    y = jnp.maximum(_dotT(h0_ref[...], w0[:, :128])
                    + _dotT(h1_ref[...], w0[:, 128:]) + b0_ref[...], 0.0)
    y = jnp.maximum(_dotT(y, w1_ref[...]) + b1_ref[...], 0.0)
    o_ref[...] = _dotT(y, w2_ref[...]) + b2_ref[...]


def _readout(h0, h1, w0, b0, w1, b1, w2, b2):
    R = 1000
    return pl.pallas_call(
        _readout_body,
        grid=(N // R,),
        in_specs=[
            pl.BlockSpec((R, 128), lambda i: (i, 0)),
            pl.BlockSpec((R, 128), lambda i: (i, 0)),
            pl.BlockSpec((128, HID), lambda i: (0, 0)),
            pl.BlockSpec((1, 128), lambda i: (0, 0)),
            pl.BlockSpec((64, 128), lambda i: (0, 0)),
            pl.BlockSpec((1, 64), lambda i: (0, 0)),
            pl.BlockSpec((2, 64), lambda i: (0, 0)),
            pl.BlockSpec((1, 2), lambda i: (0, 0)),
        ],
        out_specs=pl.BlockSpec((R, 2), lambda i: (i, 0)),
        out_shape=jax.ShapeDtypeStruct((N, 2), jnp.float32),
    )(h0, h1, w0, b0, w1, b1, w2, b2)


def _sc_mesh():
    return plsc.VectorSubcoreMesh(core_axis_name="c", subcore_axis_name="s",
                                  num_cores=_NSC, num_subcores=_NTILE)


@functools.cache
def _make_segsum():
    return functools.partial(
        pl.kernel,
        out_type=jax.ShapeDtypeStruct((_NSC, _NPAD, 128), jnp.float32),
        mesh=_sc_mesh(),
        scratch_types=[
            pltpu.VMEM_SHARED((_NPAD, 128), jnp.float32),
            pltpu.VMEM((_K,), jnp.int32),
            pltpu.VMEM((_K,), jnp.int32),
            pltpu.VMEM((_K, 128), jnp.float32),
            pltpu.SemaphoreType.DMA,
        ],
    )(_segsum_body)


def _segsum(h0, h1, src, dst, zeros):
    return _make_segsum()(h0, h1, src, dst, zeros)


def _segsum_body(h0_hbm, h1_hbm, src_hbm, dst_hbm, zeros_hbm, out_hbm,
                 acc, gidx, sidx, rows, sem):
    c = lax.axis_index("c")
    s = lax.axis_index("s")
    r0 = s * _ROWS_PER_TILE
    pltpu.sync_copy(zeros_hbm.at[pl.ds(r0, _ROWS_PER_TILE)],
                    acc.at[pl.ds(r0, _ROWS_PER_TILE)])
    plsc.subcore_barrier()

    def body(i, carry):
        ch = s + i * _NTILE

        @pl.when(ch < _NCH)
        def _():
            e0 = ch * _K
            pltpu.sync_copy(src_hbm.at[pl.ds(e0, _K)], gidx)
            pltpu.sync_copy(dst_hbm.at[pl.ds(e0, _K)], sidx)

            @pl.when(c == 0)
            def _():
                pltpu.async_copy(h0_hbm.at[gidx], rows, sem).wait()

            @pl.when(c == 1)
            def _():
                pltpu.async_copy(h1_hbm.at[gidx], rows, sem).wait()

            pltpu.sync_copy(rows, acc.at[sidx], add=True)

        return carry

    lax.fori_loop(0, (_NCH + _NTILE - 1) // _NTILE, body, 0)
    plsc.subcore_barrier()
    pltpu.sync_copy(acc.at[pl.ds(r0, _ROWS_PER_TILE)],
                    out_hbm.at[c, pl.ds(r0, _ROWS_PER_TILE)])


@functools.cache
def _make_deg():
    return functools.partial(
        pl.kernel,
        out_type=jax.ShapeDtypeStruct((_NSC, _NPAD, 128), jnp.float32),
        mesh=_sc_mesh(),
        scratch_types=[
            pltpu.VMEM_SHARED((_NPAD, 128), jnp.float32),
            pltpu.VMEM((_K,), jnp.int32),
            pltpu.VMEM((_K, 128), jnp.float32),
        ],
    )(_deg_body)


def _deg(dst, ones, zeros):
    return _make_deg()(dst, ones, zeros)


def _deg_body(dst_hbm, ones_hbm, zeros_hbm, out_hbm, acc, sidx, ones):
    c = lax.axis_index("c")
    s = lax.axis_index("s")
    r0 = s * _ROWS_PER_TILE
    pltpu.sync_copy(zeros_hbm.at[pl.ds(r0, _ROWS_PER_TILE)],
                    acc.at[pl.ds(r0, _ROWS_PER_TILE)])
    pltpu.sync_copy(ones_hbm, ones)
    plsc.subcore_barrier()
    half = _NCH // _NSC

    def body(i, carry):
        k = s + i * _NTILE

        @pl.when(k < half)
        def _():
            e0 = (c + _NSC * k) * _K
            pltpu.sync_copy(dst_hbm.at[pl.ds(e0, _K)], sidx)
            pltpu.sync_copy(ones, acc.at[sidx], add=True)

        return carry

    lax.fori_loop(0, (half + _NTILE - 1) // _NTILE, body, 0)
    plsc.subcore_barrier()
    pltpu.sync_copy(acc.at[pl.ds(r0, _ROWS_PER_TILE)],
                    out_hbm.at[c, pl.ds(r0, _ROWS_PER_TILE)])


def kernel(x, edge_index, W_emb, b_emb, W0, b0, W1, b1, W2, b2, W3, b3,
           Wm0, bm0, Wm1, bm1, Wm2, bm2):
    src = edge_index[0].astype(jnp.int32)
    dst = edge_index[1].astype(jnp.int32)
    zeros128 = jnp.zeros((_NPAD, 128), jnp.float32)
    ones128 = jnp.zeros((_K, 128), jnp.float32).at[:, 0].set(1.0)

    degp = _deg(dst, ones128, zeros128)
    h0, h1 = _emb(x, W_emb, b_emb.reshape(1, -1))
    for W, b in ((W0, b0), (W1, b1), (W2, b2), (W3, b3)):
        cs = _segsum(h0, h1, src, dst, zeros128)
        h0, h1 = _node_apply(h0, h1, cs, degp, W, b.reshape(1, -1))
    return _readout(h0, h1, Wm0, bm0.reshape(1, -1), Wm1,
                    bm1.reshape(1, -1), Wm2, bm2.reshape(1, -1))
